# Initial kernel scaffold; baseline (speedup 1.0000x reference)
import jax
import jax.numpy as jnp
from jax.experimental import pallas as pl


def kernel(x, edge_index, edge_attr, We, be, Wn, bn, Wc, bc, Wl, bl, Wo, bo):
    raise NotImplementedError("write your pallas kernel here")


# speedup vs baseline: 5.7519x; 5.7519x over previous
import jax
import jax.numpy as jnp
from jax import lax
from jax.experimental import pallas as pl
from jax.experimental.pallas import tpu as pltpu
from jax.experimental.pallas import tpu_sc as plsc

N = 10000
E = 320000
NC, NS = 2, 16
NW = NC * NS
LANE = 128
MROW = 8
EPW_ROWS = 80
E_PAD = NW * EPW_ROWS * LANE
NMACRO = EPW_ROWS // MROW
NT = 10240
TRASH = N
RPS = NT // NS
W16 = 16


def _edge_mm_body(ea_ref, w_ref, b_ref, o_ref):
    o_ref[...] = jnp.dot(ea_ref[...], w_ref[...],
                         preferred_element_type=jnp.float32) + b_ref[...]


def _edge_mm(ea128, wbd, brow):
    g = ea128.shape[0] // 4096
    return pl.pallas_call(
        _edge_mm_body,
        grid=(g,),
        in_specs=[pl.BlockSpec((4096, 128), lambda i: (i, 0)),
                  pl.BlockSpec((128, 128), lambda i: (0, 0)),
                  pl.BlockSpec((1, 128), lambda i: (0, 0))],
        out_specs=pl.BlockSpec((4096, 128), lambda i: (i, 0)),
        out_shape=jax.ShapeDtypeStruct(ea128.shape, jnp.float32),
    )(ea128, wbd, brow)


def _node_body(x_ref, wn_ref, bn_ref, wcb_ref, o_ref):
    t = jnp.maximum(jnp.dot(x_ref[...], wn_ref[...],
                            preferred_element_type=jnp.float32) + bn_ref[...], 0.0)
    o_ref[...] = jnp.dot(t, wcb_ref[...], preferred_element_type=jnp.float32)


def _node_mm(x, wn, bn_r, wcb):
    return pl.pallas_call(
        _node_body,
        grid=(10,),
        in_specs=[pl.BlockSpec((1000, 128), lambda i: (i, 0)),
                  pl.BlockSpec((128, 16), lambda i: (0, 0)),
                  pl.BlockSpec((1, 16), lambda i: (0, 0)),
                  pl.BlockSpec((16, 16), lambda i: (0, 0))],
        out_specs=pl.BlockSpec((1000, 16), lambda i: (i, 0)),
        out_shape=jax.ShapeDtypeStruct((N, W16), jnp.float32),
    )(x, wn, bn_r, wcb)


def _mid_body(a0, a1, npart, wct, o):
    acc = a0[...] + a1[...]
    norm = lax.rsqrt(jnp.maximum(acc[:, 8:9], 1.0))
    g8 = jnp.maximum(acc[:, 0:8], 0.0)
    o[...] = (jnp.dot(g8, wct[...], preferred_element_type=jnp.float32)
              + npart[...]) * norm


def _mid(acc0, acc1, npart, wct):
    spec = pl.BlockSpec((1000, 16), lambda i: (i, 0))
    return pl.pallas_call(
        _mid_body,
        grid=(10,),
        in_specs=[spec, spec, spec, pl.BlockSpec((8, 16), lambda i: (0, 0))],
        out_specs=spec,
        out_shape=jax.ShapeDtypeStruct((N, W16), jnp.float32),
    )(acc0, acc1, npart, wct)


def _head_body(g0, g1, a0, a1, bc_r, wl_r, bl_r, wo_r, bo_r, o):
    deg = a0[:, 8:9] + a1[:, 8:9]
    norm = lax.rsqrt(jnp.maximum(deg, 1.0))
    agg = (g0[...] + g1[...]) * norm
    h2 = jnp.maximum(agg + bc_r[...], 0.0)
    h3 = jnp.maximum(jnp.dot(h2, wl_r[...],
                             preferred_element_type=jnp.float32) + bl_r[...], 0.0)
    o[...] = jnp.dot(h3, wo_r[...], preferred_element_type=jnp.float32) + bo_r[...]


def _head(agg0, agg1, acc0, acc1, bc_r, wl_r, bl_r, wo_r, bo_r):
    spec = pl.BlockSpec((1000, 16), lambda i: (i, 0))
    w16spec = pl.BlockSpec((16, 16), lambda i: (0, 0))
    b16spec = pl.BlockSpec((1, 16), lambda i: (0, 0))
    return pl.pallas_call(
        _head_body,
        grid=(10,),
        in_specs=[spec, spec, spec, spec, b16spec, w16spec, b16spec,
                  pl.BlockSpec((16, 2), lambda i: (0, 0)),
                  pl.BlockSpec((1, 2), lambda i: (0, 0))],
        out_specs=pl.BlockSpec((1000, 2), lambda i: (i, 0)),
        out_shape=jax.ShapeDtypeStruct((N, 2), jnp.float32),
    )(agg0, agg1, acc0, acc1, bc_r, wl_r, bl_r, wo_r, bo_r)


_MESH = plsc.VectorSubcoreMesh(core_axis_name="c", subcore_axis_name="s",
                               num_cores=NC, num_subcores=NS)


def _sc_scatter_body(em_hbm, dst_hbm, zero_hbm, out_hbm, idx_v, rows_v, table):
    c = lax.axis_index("c")
    s = lax.axis_index("s")
    w = c * NS + s

    @pl.when(s == 0)
    def _init():
        pltpu.sync_copy(zero_hbm, table)

    plsc.subcore_barrier()

    def macro(m, carry):
        r0 = w * EPW_ROWS + m * MROW
        pltpu.sync_copy(dst_hbm.at[pl.ds(r0, MROW)], idx_v)
        pltpu.sync_copy(em_hbm.at[pl.ds(r0 * LANE, MROW * LANE)], rows_v)
        for j in range(MROW):
            pltpu.sync_copy(rows_v.at[pl.ds(j * LANE, LANE)],
                            table.at[idx_v.at[j]], add=True)
        return carry

    lax.fori_loop(0, NMACRO, macro, 0)
    plsc.subcore_barrier()
    pltpu.sync_copy(table.at[pl.ds(s * RPS, RPS)],
                    out_hbm.at[c].at[pl.ds(s * RPS, RPS)])


_sc_scatter = pl.kernel(
    _sc_scatter_body,
    out_type=jax.ShapeDtypeStruct((NC, NT, W16), jnp.float32),
    mesh=_MESH,
    scratch_types=[pltpu.VMEM((MROW, LANE), jnp.int32),
                   pltpu.VMEM((MROW * LANE, W16), jnp.float32),
                   pltpu.VMEM_SHARED((NT, W16), jnp.float32)],
    compiler_params=pltpu.CompilerParams(use_tc_tiling_on_sc=False),
)


def _sc_gs_body(g_hbm, src_hbm, dst_hbm, zero_hbm, out_hbm,
                sidx_v, didx_v, rows_v, table, sem):
    c = lax.axis_index("c")
    s = lax.axis_index("s")
    w = c * NS + s

    @pl.when(s == 0)
    def _init():
        pltpu.sync_copy(zero_hbm, table)

    plsc.subcore_barrier()

    def macro(m, carry):
        r0 = w * EPW_ROWS + m * MROW
        pltpu.sync_copy(src_hbm.at[pl.ds(r0, MROW)], sidx_v)
        pltpu.sync_copy(dst_hbm.at[pl.ds(r0, MROW)], didx_v)
        for j in range(MROW):
            pltpu.async_copy(g_hbm.at[sidx_v.at[j]],
                             rows_v.at[pl.ds(j * LANE, LANE)], sem).wait()
        for j in range(MROW):
            pltpu.sync_copy(rows_v.at[pl.ds(j * LANE, LANE)],
                            table.at[didx_v.at[j]], add=True)
        return carry

    lax.fori_loop(0, NMACRO, macro, 0)
    plsc.subcore_barrier()
    pltpu.sync_copy(table.at[pl.ds(s * RPS, RPS)],
                    out_hbm.at[c].at[pl.ds(s * RPS, RPS)])


_sc_gather_scatter = pl.kernel(
    _sc_gs_body,
    out_type=jax.ShapeDtypeStruct((NC, NT, W16), jnp.float32),
    mesh=_MESH,
    scratch_types=[pltpu.VMEM((MROW, LANE), jnp.int32),
                   pltpu.VMEM((MROW, LANE), jnp.int32),
                   pltpu.VMEM((MROW * LANE, W16), jnp.float32),
                   pltpu.VMEM_SHARED((NT, W16), jnp.float32),
                   pltpu.SemaphoreType.DMA],
    compiler_params=pltpu.CompilerParams(use_tc_tiling_on_sc=False),
)


def kernel(x, edge_index, edge_attr, We, be, Wn, bn, Wc, bc, Wl, bl, Wo, bo):
    f32 = jnp.float32
    src = edge_index[0].astype(jnp.int32)
    dst = edge_index[1].astype(jnp.int32)
    pad = E_PAD - E
    src3 = jnp.concatenate([src, jnp.zeros((pad,), jnp.int32)]
                           ).reshape(NW * EPW_ROWS, LANE)
    dst3 = jnp.concatenate([dst, jnp.full((pad,), TRASH, jnp.int32)]
                           ).reshape(NW * EPW_ROWS, LANE)
    ea_pad = jnp.concatenate([edge_attr, jnp.zeros((pad, 16), f32)], axis=0)
    ea128 = ea_pad.reshape(E_PAD // 8, 128)

    We16 = jnp.zeros((16, 16), f32).at[:, :8].set(We)
    wbd = jnp.kron(jnp.eye(8, dtype=f32), We16)
    brow = jnp.tile(jnp.concatenate([be, jnp.ones((1,), f32),
                                     jnp.zeros((7,), f32)]), 8).reshape(1, 128)
    wct = jnp.zeros((8, 16), f32).at[:, :10].set(Wc[:8])
    wcb = jnp.zeros((16, 16), f32).at[:, :10].set(Wc[8:])
    bn_r = bn.reshape(1, 16)
    bc_r = jnp.zeros((1, 16), f32).at[0, :10].set(bc)
    wl_r = jnp.zeros((16, 16), f32).at[:10, :10].set(Wl)
    bl_r = jnp.zeros((1, 16), f32).at[0, :10].set(bl)
    wo_r = jnp.zeros((16, 2), f32).at[:10].set(Wo)
    bo_r = bo.reshape(1, 2)
    zeros_t = jnp.zeros((NT, W16), f32)

    em16 = _edge_mm(ea128, wbd, brow).reshape(E_PAD, W16)
    npart = _node_mm(x, Wn, bn_r, wcb)
    accp = _sc_scatter(em16, dst3, zeros_t)
    acc0, acc1 = accp[0, :N], accp[1, :N]
    gsc = _mid(acc0, acc1, npart, wct)
    aggp = _sc_gather_scatter(gsc, src3, dst3, zeros_t)
    return _head(aggp[0, :N], aggp[1, :N], acc0, acc1,
                 bc_r, wl_r, bl_r, wo_r, bo_r)


# speedup vs baseline: 7.0301x; 1.2222x over previous
import jax
import jax.numpy as jnp
from jax import lax
from jax.experimental import pallas as pl
from jax.experimental.pallas import tpu as pltpu
from jax.experimental.pallas import tpu_sc as plsc

N = 10000
E = 320000
NC, NS = 2, 16
NW = NC * NS
LANE = 128
MROW = 8
EPW_ROWS = 80
E_PAD = NW * EPW_ROWS * LANE
NMACRO = EPW_ROWS // MROW
NT = 10240
TRASH = N
RPS = NT // NS
W16 = 16


def _front_body(ea_ref, x_ref, w_ref, b_ref, wn_ref, bn_ref, wcb_ref,
                em_ref, np_ref):
    em_ref[...] = jnp.dot(ea_ref[...], w_ref[...],
                          preferred_element_type=jnp.float32) + b_ref[...]
    t = jnp.maximum(jnp.dot(x_ref[...], wn_ref[...],
                            preferred_element_type=jnp.float32) + bn_ref[...], 0.0)
    np_ref[...] = jnp.dot(t, wcb_ref[...], preferred_element_type=jnp.float32)


def _front(ea128, x, wbd, brow, wn, bn_r, wcb):
    return pl.pallas_call(
        _front_body,
        grid=(10,),
        in_specs=[pl.BlockSpec((E_PAD // 80, 128), lambda i: (i, 0)),
                  pl.BlockSpec((1000, 128), lambda i: (i, 0)),
                  pl.BlockSpec((128, 128), lambda i: (0, 0)),
                  pl.BlockSpec((1, 128), lambda i: (0, 0)),
                  pl.BlockSpec((128, 16), lambda i: (0, 0)),
                  pl.BlockSpec((1, 16), lambda i: (0, 0)),
                  pl.BlockSpec((16, 16), lambda i: (0, 0))],
        out_specs=[pl.BlockSpec((E_PAD // 80, 128), lambda i: (i, 0)),
                   pl.BlockSpec((1000, 16), lambda i: (i, 0))],
        out_shape=[jax.ShapeDtypeStruct((E_PAD // 8, 128), jnp.float32),
                   jax.ShapeDtypeStruct((N, W16), jnp.float32)],
    )(ea128, x, wbd, brow, wn, bn_r, wcb)


def _mid_body(a0, a1, npart, wct, o):
    acc = a0[...] + a1[...]
    norm = lax.rsqrt(jnp.maximum(acc[:, 8:9], 1.0))
    g8 = jnp.maximum(acc[:, 0:8], 0.0)
    o[...] = (jnp.dot(g8, wct[...], preferred_element_type=jnp.float32)
              + npart[...]) * norm


def _mid(acc0, acc1, npart, wct):
    spec = pl.BlockSpec((1000, 16), lambda i: (i, 0))
    return pl.pallas_call(
        _mid_body,
        grid=(10,),
        in_specs=[spec, spec, spec, pl.BlockSpec((8, 16), lambda i: (0, 0))],
        out_specs=spec,
        out_shape=jax.ShapeDtypeStruct((N, W16), jnp.float32),
    )(acc0, acc1, npart, wct)


def _head_body(g0, g1, a0, a1, bc_r, wl_r, bl_r, wo_r, bo_r, o):
    deg = a0[:, 8:9] + a1[:, 8:9]
    norm = lax.rsqrt(jnp.maximum(deg, 1.0))
    agg = (g0[...] + g1[...]) * norm
    h2 = jnp.maximum(agg + bc_r[...], 0.0)
    h3 = jnp.maximum(jnp.dot(h2, wl_r[...],
                             preferred_element_type=jnp.float32) + bl_r[...], 0.0)
    o[...] = jnp.dot(h3, wo_r[...], preferred_element_type=jnp.float32) + bo_r[...]


def _head(agg0, agg1, acc0, acc1, bc_r, wl_r, bl_r, wo_r, bo_r):
    spec = pl.BlockSpec((1000, 16), lambda i: (i, 0))
    w16spec = pl.BlockSpec((16, 16), lambda i: (0, 0))
    b16spec = pl.BlockSpec((1, 16), lambda i: (0, 0))
    return pl.pallas_call(
        _head_body,
        grid=(10,),
        in_specs=[spec, spec, spec, spec, b16spec, w16spec, b16spec,
                  pl.BlockSpec((16, 2), lambda i: (0, 0)),
                  pl.BlockSpec((1, 2), lambda i: (0, 0))],
        out_specs=pl.BlockSpec((1000, 2), lambda i: (i, 0)),
        out_shape=jax.ShapeDtypeStruct((N, 2), jnp.float32),
    )(agg0, agg1, acc0, acc1, bc_r, wl_r, bl_r, wo_r, bo_r)


_MESH = plsc.VectorSubcoreMesh(core_axis_name="c", subcore_axis_name="s",
                               num_cores=NC, num_subcores=NS)


def _sc_scatter_body(em_hbm, dst_hbm, zero_hbm, out_hbm, idx_v, rows_v, table,
                     sem2):
    c = lax.axis_index("c")
    s = lax.axis_index("s")
    w = c * NS + s

    @pl.when(s == 0)
    def _init():
        pltpu.sync_copy(zero_hbm, table)

    plsc.subcore_barrier()

    def macro(m, carry):
        r0 = w * EPW_ROWS + m * MROW
        pltpu.sync_copy(dst_hbm.at[pl.ds(r0, MROW)], idx_v)
        pltpu.sync_copy(em_hbm.at[pl.ds(r0 * LANE, MROW * LANE)], rows_v)
        puts = [pltpu.async_copy(rows_v.at[pl.ds(j * LANE, LANE)],
                                 table.at[idx_v.at[j]], sem2, add=True)
                for j in range(MROW)]
        for p in puts:
            p.wait()
        return carry

    lax.fori_loop(0, NMACRO, macro, 0)
    plsc.subcore_barrier()
    pltpu.sync_copy(table.at[pl.ds(s * RPS, RPS)],
                    out_hbm.at[c].at[pl.ds(s * RPS, RPS)])


_sc_scatter = pl.kernel(
    _sc_scatter_body,
    out_type=jax.ShapeDtypeStruct((NC, NT, W16), jnp.float32),
    mesh=_MESH,
    scratch_types=[pltpu.VMEM((MROW, LANE), jnp.int32),
                   pltpu.VMEM((MROW * LANE, W16), jnp.float32),
                   pltpu.VMEM_SHARED((NT, W16), jnp.float32),
                   pltpu.SemaphoreType.DMA],
    compiler_params=pltpu.CompilerParams(use_tc_tiling_on_sc=False),
)


def _sc_gs_body(g_hbm, src_hbm, dst_hbm, zero_hbm, out_hbm,
                sidx_v, didx_v, rows_v, tg, table, sem, sem2):
    c = lax.axis_index("c")
    s = lax.axis_index("s")
    w = c * NS + s

    @pl.when(s == 0)
    def _init():
        pltpu.sync_copy(zero_hbm, table)

    @pl.when(s < 10)
    def _stage():
        pltpu.sync_copy(g_hbm.at[pl.ds(s * 1000, 1000)],
                        tg.at[pl.ds(s * 1000, 1000)])

    plsc.subcore_barrier()

    def macro(m, carry):
        r0 = w * EPW_ROWS + m * MROW
        pltpu.sync_copy(src_hbm.at[pl.ds(r0, MROW)], sidx_v)
        pltpu.sync_copy(dst_hbm.at[pl.ds(r0, MROW)], didx_v)
        gets = [pltpu.async_copy(tg.at[sidx_v.at[j]],
                                 rows_v.at[pl.ds(j * LANE, LANE)], sem)
                for j in range(MROW)]
        puts = []
        for j in range(MROW):
            gets[j].wait()
            puts.append(pltpu.async_copy(rows_v.at[pl.ds(j * LANE, LANE)],
                                         table.at[didx_v.at[j]], sem2,
                                         add=True))
        for p in puts:
            p.wait()
        return carry

    lax.fori_loop(0, NMACRO, macro, 0)
    plsc.subcore_barrier()
    pltpu.sync_copy(table.at[pl.ds(s * RPS, RPS)],
                    out_hbm.at[c].at[pl.ds(s * RPS, RPS)])


_sc_gather_scatter = pl.kernel(
    _sc_gs_body,
    out_type=jax.ShapeDtypeStruct((NC, NT, W16), jnp.float32),
    mesh=_MESH,
    scratch_types=[pltpu.VMEM((MROW, LANE), jnp.int32),
                   pltpu.VMEM((MROW, LANE), jnp.int32),
                   pltpu.VMEM((MROW * LANE, W16), jnp.float32),
                   pltpu.VMEM_SHARED((N, W16), jnp.float32),
                   pltpu.VMEM_SHARED((NT, W16), jnp.float32),
                   pltpu.SemaphoreType.DMA,
                   pltpu.SemaphoreType.DMA],
    compiler_params=pltpu.CompilerParams(use_tc_tiling_on_sc=False),
)


def kernel(x, edge_index, edge_attr, We, be, Wn, bn, Wc, bc, Wl, bl, Wo, bo):
    f32 = jnp.float32
    src = edge_index[0].astype(jnp.int32)
    dst = edge_index[1].astype(jnp.int32)
    pad = E_PAD - E
    src3 = jnp.concatenate([src, jnp.zeros((pad,), jnp.int32)]
                           ).reshape(NW * EPW_ROWS, LANE)
    dst3 = jnp.concatenate([dst, jnp.full((pad,), TRASH, jnp.int32)]
                           ).reshape(NW * EPW_ROWS, LANE)
    ea_pad = jnp.concatenate([edge_attr, jnp.zeros((pad, 16), f32)], axis=0)
    ea128 = ea_pad.reshape(E_PAD // 8, 128)

    We16 = jnp.zeros((16, 16), f32).at[:, :8].set(We)
    wbd = jnp.kron(jnp.eye(8, dtype=f32), We16)
    brow = jnp.tile(jnp.concatenate([be, jnp.ones((1,), f32),
                                     jnp.zeros((7,), f32)]), 8).reshape(1, 128)
    wct = jnp.zeros((8, 16), f32).at[:, :10].set(Wc[:8])
    wcb = jnp.zeros((16, 16), f32).at[:, :10].set(Wc[8:])
    bn_r = bn.reshape(1, 16)
    bc_r = jnp.zeros((1, 16), f32).at[0, :10].set(bc)
    wl_r = jnp.zeros((16, 16), f32).at[:10, :10].set(Wl)
    bl_r = jnp.zeros((1, 16), f32).at[0, :10].set(bl)
    wo_r = jnp.zeros((16, 2), f32).at[:10].set(Wo)
    bo_r = bo.reshape(1, 2)
    zeros_t = jnp.zeros((NT, W16), f32)

    em128, npart = _front(ea128, x, wbd, brow, Wn, bn_r, wcb)
    em16 = em128.reshape(E_PAD, W16)
    accp = _sc_scatter(em16, dst3, zeros_t)
    acc0, acc1 = accp[0, :N], accp[1, :N]
    gsc = _mid(acc0, acc1, npart, wct)
    aggp = _sc_gather_scatter(gsc, src3, dst3, zeros_t)
    return _head(aggp[0, :N], aggp[1, :N], acc0, acc1,
                 bc_r, wl_r, bl_r, wo_r, bo_r)


# speedup vs baseline: 8.7564x; 1.2456x over previous
import jax
import jax.numpy as jnp
from jax import lax
from jax.experimental import pallas as pl
from jax.experimental.pallas import tpu as pltpu
from jax.experimental.pallas import tpu_sc as plsc

N = 10000
E = 320000
NC, NS = 2, 16
NW = NC * NS
LANE = 80
MROW = 5
RPW = E // (NW * LANE)
NMACRO = RPW // MROW
NT = 10240
RPS = NT // NS
W16 = 16


def _front_body(x_ref, wn_ref, bn_ref, wcb_ref, np_ref):
    t = jnp.maximum(jnp.dot(x_ref[...], wn_ref[...],
                            preferred_element_type=jnp.float32) + bn_ref[...], 0.0)
    np_ref[...] = jnp.dot(t, wcb_ref[...], preferred_element_type=jnp.float32)


def _front(x, wn, bn_r, wcb):
    return pl.pallas_call(
        _front_body,
        grid=(10,),
        in_specs=[pl.BlockSpec((1000, 128), lambda i: (i, 0)),
                  pl.BlockSpec((128, 16), lambda i: (0, 0)),
                  pl.BlockSpec((1, 16), lambda i: (0, 0)),
                  pl.BlockSpec((16, 16), lambda i: (0, 0))],
        out_specs=pl.BlockSpec((1000, 16), lambda i: (i, 0)),
        out_shape=jax.ShapeDtypeStruct((N, W16), jnp.float32),
    )(x, wn, bn_r, wcb)


def _mid_body(a0, a1, d0, d1, npart, we16, be16, wct, o):
    acc = a0[...] + a1[...]
    deg = d0[:, 0:1] + d1[:, 0:1]
    norm = lax.rsqrt(jnp.maximum(deg, 1.0))
    eacc = jnp.dot(acc, we16[...], preferred_element_type=jnp.float32) \
        + deg * be16[...]
    eh = jnp.maximum(eacc, 0.0)
    g = (jnp.dot(eh, wct[...], preferred_element_type=jnp.float32)
         + npart[...]) * norm
    col = lax.broadcasted_iota(jnp.int32, g.shape, 1)
    o[...] = jnp.where(col == 15, 1.0, g)


def _mid(acc0, acc1, deg0, deg1, npart, we16, be16, wct):
    spec = pl.BlockSpec((1000, 16), lambda i: (i, 0))
    w16spec = pl.BlockSpec((16, 16), lambda i: (0, 0))
    return pl.pallas_call(
        _mid_body,
        grid=(10,),
        in_specs=[spec, spec, spec, spec, spec, w16spec,
                  pl.BlockSpec((1, 16), lambda i: (0, 0)), w16spec],
        out_specs=spec,
        out_shape=jax.ShapeDtypeStruct((N, W16), jnp.float32),
    )(acc0, acc1, deg0, deg1, npart, we16, be16, wct)


def _head_body(g0, g1, bc_r, wl_r, bl_r, wo_r, bo_r, o):
    deg = g0[:, 15:16] + g1[:, 15:16]
    norm = lax.rsqrt(jnp.maximum(deg, 1.0))
    agg = (g0[...] + g1[...]) * norm
    h2 = jnp.maximum(agg + bc_r[...], 0.0)
    h3 = jnp.maximum(jnp.dot(h2, wl_r[...],
                             preferred_element_type=jnp.float32) + bl_r[...], 0.0)
    o[...] = jnp.dot(h3, wo_r[...], preferred_element_type=jnp.float32) + bo_r[...]


def _head(agg0, agg1, bc_r, wl_r, bl_r, wo_r, bo_r):
    spec = pl.BlockSpec((1000, 16), lambda i: (i, 0))
    b16spec = pl.BlockSpec((1, 16), lambda i: (0, 0))
    return pl.pallas_call(
        _head_body,
        grid=(10,),
        in_specs=[spec, spec, b16spec,
                  pl.BlockSpec((16, 16), lambda i: (0, 0)), b16spec,
                  pl.BlockSpec((16, 2), lambda i: (0, 0)),
                  pl.BlockSpec((1, 2), lambda i: (0, 0))],
        out_specs=pl.BlockSpec((1000, 2), lambda i: (i, 0)),
        out_shape=jax.ShapeDtypeStruct((N, 2), jnp.float32),
    )(agg0, agg1, bc_r, wl_r, bl_r, wo_r, bo_r)


_MESH = plsc.VectorSubcoreMesh(core_axis_name="c", subcore_axis_name="s",
                               num_cores=NC, num_subcores=NS)


def _sc_scatter_body(ea_hbm, dst_hbm, ones_hbm, zero_hbm, out_hbm, deg_hbm,
                     idx_v, rows_v, ones_v, table, tdeg, sem2):
    c = lax.axis_index("c")
    s = lax.axis_index("s")
    w = c * NS + s

    @pl.when(s == 0)
    def _init():
        pltpu.sync_copy(zero_hbm, table)

    @pl.when(s == 1)
    def _init2():
        pltpu.sync_copy(zero_hbm, tdeg)

    pltpu.sync_copy(ones_hbm, ones_v)
    plsc.subcore_barrier()

    def macro(m, carry):
        r0 = w * RPW + m * MROW
        pltpu.sync_copy(dst_hbm.at[pl.ds(r0, MROW)], idx_v)
        pltpu.sync_copy(ea_hbm.at[pl.ds(r0 * LANE, MROW * LANE)], rows_v)
        puts = []
        for j in range(MROW):
            puts.append(pltpu.async_copy(rows_v.at[pl.ds(j * LANE, LANE)],
                                         table.at[idx_v.at[j]], sem2,
                                         add=True))
            puts.append(pltpu.async_copy(ones_v,
                                         tdeg.at[idx_v.at[j]], sem2,
                                         add=True))
        for p in puts:
            p.wait()
        return carry

    lax.fori_loop(0, NMACRO, macro, 0)
    plsc.subcore_barrier()
    pltpu.sync_copy(table.at[pl.ds(s * RPS, RPS)],
                    out_hbm.at[c].at[pl.ds(s * RPS, RPS)])
    pltpu.sync_copy(tdeg.at[pl.ds(s * RPS, RPS)],
                    deg_hbm.at[c].at[pl.ds(s * RPS, RPS)])


_sc_scatter = pl.kernel(
    _sc_scatter_body,
    out_type=(jax.ShapeDtypeStruct((NC, NT, W16), jnp.float32),
              jax.ShapeDtypeStruct((NC, NT, W16), jnp.float32)),
    mesh=_MESH,
    scratch_types=[pltpu.VMEM((MROW, LANE), jnp.int32),
                   pltpu.VMEM((MROW * LANE, W16), jnp.float32),
                   pltpu.VMEM((LANE, W16), jnp.float32),
                   pltpu.VMEM_SHARED((NT, W16), jnp.float32),
                   pltpu.VMEM_SHARED((NT, W16), jnp.float32),
                   pltpu.SemaphoreType.DMA],
    compiler_params=pltpu.CompilerParams(use_tc_tiling_on_sc=False),
)


def _sc_gs_body(g_hbm, src_hbm, dst_hbm, zero_hbm, out_hbm,
                sidx_v, didx_v, rows_v, tg, table, sem, sem2):
    c = lax.axis_index("c")
    s = lax.axis_index("s")
    w = c * NS + s

    @pl.when(s == 0)
    def _init():
        pltpu.sync_copy(zero_hbm, table)

    @pl.when(s < 10)
    def _stage():
        pltpu.sync_copy(g_hbm.at[pl.ds(s * 1000, 1000)],
                        tg.at[pl.ds(s * 1000, 1000)])

    plsc.subcore_barrier()

    def macro(m, carry):
        r0 = w * RPW + m * MROW
        pltpu.sync_copy(src_hbm.at[pl.ds(r0, MROW)], sidx_v)
        pltpu.sync_copy(dst_hbm.at[pl.ds(r0, MROW)], didx_v)
        gets = [pltpu.async_copy(tg.at[sidx_v.at[j]],
                                 rows_v.at[pl.ds(j * LANE, LANE)], sem)
                for j in range(MROW)]
        puts = []
        for j in range(MROW):
            gets[j].wait()
            puts.append(pltpu.async_copy(rows_v.at[pl.ds(j * LANE, LANE)],
                                         table.at[didx_v.at[j]], sem2,
                                         add=True))
        for p in puts:
            p.wait()
        return carry

    lax.fori_loop(0, NMACRO, macro, 0)
    plsc.subcore_barrier()
    pltpu.sync_copy(table.at[pl.ds(s * RPS, RPS)],
                    out_hbm.at[c].at[pl.ds(s * RPS, RPS)])


_sc_gather_scatter = pl.kernel(
    _sc_gs_body,
    out_type=jax.ShapeDtypeStruct((NC, NT, W16), jnp.float32),
    mesh=_MESH,
    scratch_types=[pltpu.VMEM((MROW, LANE), jnp.int32),
                   pltpu.VMEM((MROW, LANE), jnp.int32),
                   pltpu.VMEM((MROW * LANE, W16), jnp.float32),
                   pltpu.VMEM_SHARED((N, W16), jnp.float32),
                   pltpu.VMEM_SHARED((NT, W16), jnp.float32),
                   pltpu.SemaphoreType.DMA,
                   pltpu.SemaphoreType.DMA],
    compiler_params=pltpu.CompilerParams(use_tc_tiling_on_sc=False),
)


def kernel(x, edge_index, edge_attr, We, be, Wn, bn, Wc, bc, Wl, bl, Wo, bo):
    f32 = jnp.float32
    src80 = edge_index[0].astype(jnp.int32).reshape(E // LANE, LANE)
    dst80 = edge_index[1].astype(jnp.int32).reshape(E // LANE, LANE)

    we16 = jnp.zeros((16, 16), f32).at[:, :8].set(We)
    be16 = jnp.zeros((1, 16), f32).at[0, :8].set(be)
    wct = jnp.zeros((16, 16), f32).at[:8, :10].set(Wc[:8])
    wcb = jnp.zeros((16, 16), f32).at[:, :10].set(Wc[8:])
    bn_r = bn.reshape(1, 16)
    bc_r = jnp.zeros((1, 16), f32).at[0, :10].set(bc)
    wl_r = jnp.zeros((16, 16), f32).at[:10, :10].set(Wl)
    bl_r = jnp.zeros((1, 16), f32).at[0, :10].set(bl)
    wo_r = jnp.zeros((16, 2), f32).at[:10].set(Wo)
    bo_r = bo.reshape(1, 2)
    zeros_t = jnp.zeros((NT, W16), f32)
    ones_b = jnp.ones((LANE, W16), f32)

    npart = _front(x, Wn, bn_r, wcb)
    accp, degp = _sc_scatter(edge_attr, dst80, ones_b, zeros_t)
    gsc = _mid(accp[0, :N], accp[1, :N], degp[0, :N], degp[1, :N],
               npart, we16, be16, wct)
    aggp = _sc_gather_scatter(gsc, src80, dst80, zeros_t)
    return _head(aggp[0, :N], aggp[1, :N], bc_r, wl_r, bl_r, wo_r, bo_r)


# speedup vs baseline: 9.3817x; 1.0714x over previous
import jax
import jax.numpy as jnp
from jax import lax
from jax.experimental import pallas as pl
from jax.experimental.pallas import tpu as pltpu
from jax.experimental.pallas import tpu_sc as plsc

N = 10000
E = 320000
NC, NS = 2, 16
NW = NC * NS
LANE = 128
MROW = 6
NROWS = E // LANE
RPW = 78
NMACRO = RPW // MROW
XBASE = NW * RPW
NT = 10240
RPS = NT // NS
W16 = 16


def _front_body(x_ref, wn_ref, bn_ref, wcb_ref, np_ref):
    t = jnp.maximum(jnp.dot(x_ref[...], wn_ref[...],
                            preferred_element_type=jnp.float32) + bn_ref[...], 0.0)
    np_ref[...] = jnp.dot(t, wcb_ref[...], preferred_element_type=jnp.float32)


def _front(x, wn, bn_r, wcb):
    return pl.pallas_call(
        _front_body,
        grid=(10,),
        in_specs=[pl.BlockSpec((1000, 128), lambda i: (i, 0)),
                  pl.BlockSpec((128, 16), lambda i: (0, 0)),
                  pl.BlockSpec((1, 16), lambda i: (0, 0)),
                  pl.BlockSpec((16, 16), lambda i: (0, 0))],
        out_specs=pl.BlockSpec((1000, 16), lambda i: (i, 0)),
        out_shape=jax.ShapeDtypeStruct((N, W16), jnp.float32),
    )(x, wn, bn_r, wcb)


def _mid_body(a0, a1, d0, d1, npart, we16, be16, wct, o):
    acc = a0[...] + a1[...]
    deg = d0[:, 0:1] + d1[:, 0:1]
    norm = lax.rsqrt(jnp.maximum(deg, 1.0))
    eacc = jnp.dot(acc, we16[...], preferred_element_type=jnp.float32) \
        + deg * be16[...]
    eh = jnp.maximum(eacc, 0.0)
    g = (jnp.dot(eh, wct[...], preferred_element_type=jnp.float32)
         + npart[...]) * norm
    col = lax.broadcasted_iota(jnp.int32, g.shape, 1)
    o[...] = jnp.where(col == 15, 1.0, g)


def _mid(acc0, acc1, deg0, deg1, npart, we16, be16, wct):
    spec = pl.BlockSpec((1000, 16), lambda i: (i, 0))
    w16spec = pl.BlockSpec((16, 16), lambda i: (0, 0))
    return pl.pallas_call(
        _mid_body,
        grid=(10,),
        in_specs=[spec, spec, spec, spec, spec, w16spec,
                  pl.BlockSpec((1, 16), lambda i: (0, 0)), w16spec],
        out_specs=spec,
        out_shape=jax.ShapeDtypeStruct((N, W16), jnp.float32),
    )(acc0, acc1, deg0, deg1, npart, we16, be16, wct)


def _head_body(g0, g1, bc_r, wl_r, bl_r, wo_r, bo_r, o):
    deg = g0[:, 15:16] + g1[:, 15:16]
    norm = lax.rsqrt(jnp.maximum(deg, 1.0))
    agg = (g0[...] + g1[...]) * norm
    h2 = jnp.maximum(agg + bc_r[...], 0.0)
    h3 = jnp.maximum(jnp.dot(h2, wl_r[...],
                             preferred_element_type=jnp.float32) + bl_r[...], 0.0)
    o[...] = jnp.dot(h3, wo_r[...], preferred_element_type=jnp.float32) + bo_r[...]


def _head(agg0, agg1, bc_r, wl_r, bl_r, wo_r, bo_r):
    spec = pl.BlockSpec((1000, 16), lambda i: (i, 0))
    b16spec = pl.BlockSpec((1, 16), lambda i: (0, 0))
    return pl.pallas_call(
        _head_body,
        grid=(10,),
        in_specs=[spec, spec, b16spec,
                  pl.BlockSpec((16, 16), lambda i: (0, 0)), b16spec,
                  pl.BlockSpec((16, 2), lambda i: (0, 0)),
                  pl.BlockSpec((1, 2), lambda i: (0, 0))],
        out_specs=pl.BlockSpec((1000, 2), lambda i: (i, 0)),
        out_shape=jax.ShapeDtypeStruct((N, 2), jnp.float32),
    )(agg0, agg1, bc_r, wl_r, bl_r, wo_r, bo_r)


_MESH = plsc.VectorSubcoreMesh(core_axis_name="c", subcore_axis_name="s",
                               num_cores=NC, num_subcores=NS)


def _sc_scatter_body(ea_hbm, dst_hbm, ones_hbm, zero_hbm, out_hbm, deg_hbm,
                     idx_v, rows_v, ones_v, table, tdeg, sem2):
    c = lax.axis_index("c")
    s = lax.axis_index("s")
    w = c * NS + s

    @pl.when(s == 0)
    def _init():
        pltpu.sync_copy(zero_hbm, table)

    @pl.when(s == 1)
    def _init2():
        pltpu.sync_copy(zero_hbm, tdeg)

    pltpu.sync_copy(ones_hbm, ones_v)
    plsc.subcore_barrier()

    def macro(m, carry):
        r0 = w * RPW + m * MROW
        pltpu.sync_copy(dst_hbm.at[pl.ds(r0, MROW)], idx_v)
        pltpu.sync_copy(ea_hbm.at[pl.ds(r0 * LANE, MROW * LANE)], rows_v)
        puts = []
        for j in range(MROW):
            puts.append(pltpu.async_copy(rows_v.at[pl.ds(j * LANE, LANE)],
                                         table.at[idx_v.at[j]], sem2,
                                         add=True))
            puts.append(pltpu.async_copy(ones_v,
                                         tdeg.at[idx_v.at[j]], sem2,
                                         add=True))
        for p in puts:
            p.wait()
        return carry

    lax.fori_loop(0, NMACRO, macro, 0)

    @pl.when(w < NROWS - XBASE)
    def _leftover():
        r0 = XBASE + w
        pltpu.sync_copy(dst_hbm.at[pl.ds(r0, 1)], idx_v.at[pl.ds(0, 1)])
        pltpu.sync_copy(ea_hbm.at[pl.ds(r0 * LANE, LANE)],
                        rows_v.at[pl.ds(0, LANE)])
        p1 = pltpu.async_copy(rows_v.at[pl.ds(0, LANE)],
                              table.at[idx_v.at[0]], sem2, add=True)
        p2 = pltpu.async_copy(ones_v, tdeg.at[idx_v.at[0]], sem2, add=True)
        p1.wait()
        p2.wait()

    plsc.subcore_barrier()
    pltpu.sync_copy(table.at[pl.ds(s * RPS, RPS)],
                    out_hbm.at[c].at[pl.ds(s * RPS, RPS)])
    pltpu.sync_copy(tdeg.at[pl.ds(s * RPS, RPS)],
                    deg_hbm.at[c].at[pl.ds(s * RPS, RPS)])


_sc_scatter = pl.kernel(
    _sc_scatter_body,
    out_type=(jax.ShapeDtypeStruct((NC, NT, W16), jnp.float32),
              jax.ShapeDtypeStruct((NC, NT, W16), jnp.float32)),
    mesh=_MESH,
    scratch_types=[pltpu.VMEM((MROW, LANE), jnp.int32),
                   pltpu.VMEM((MROW * LANE, W16), jnp.float32),
                   pltpu.VMEM((LANE, W16), jnp.float32),
                   pltpu.VMEM_SHARED((NT, W16), jnp.float32),
                   pltpu.VMEM_SHARED((NT, W16), jnp.float32),
                   pltpu.SemaphoreType.DMA],
    compiler_params=pltpu.CompilerParams(use_tc_tiling_on_sc=False),
)


def _sc_gs_body(g_hbm, src_hbm, dst_hbm, zero_hbm, out_hbm,
                sidx_v, didx_v, rows_v, tg, table, sem, sem2):
    c = lax.axis_index("c")
    s = lax.axis_index("s")
    w = c * NS + s

    @pl.when(s == 0)
    def _init():
        pltpu.sync_copy(zero_hbm, table)

    @pl.when(s < 10)
    def _stage():
        pltpu.sync_copy(g_hbm.at[pl.ds(s * 1000, 1000)],
                        tg.at[pl.ds(s * 1000, 1000)])

    plsc.subcore_barrier()

    def macro(m, carry):
        r0 = w * RPW + m * MROW
        pltpu.sync_copy(src_hbm.at[pl.ds(r0, MROW)], sidx_v)
        pltpu.sync_copy(dst_hbm.at[pl.ds(r0, MROW)], didx_v)
        gets = [pltpu.async_copy(tg.at[sidx_v.at[j]],
                                 rows_v.at[pl.ds(j * LANE, LANE)], sem)
                for j in range(MROW)]
        puts = []
        for j in range(MROW):
            gets[j].wait()
            puts.append(pltpu.async_copy(rows_v.at[pl.ds(j * LANE, LANE)],
                                         table.at[didx_v.at[j]], sem2,
                                         add=True))
        for p in puts:
            p.wait()
        return carry

    lax.fori_loop(0, NMACRO, macro, 0)

    @pl.when(w < NROWS - XBASE)
    def _leftover():
        r0 = XBASE + w
        pltpu.sync_copy(src_hbm.at[pl.ds(r0, 1)], sidx_v.at[pl.ds(0, 1)])
        pltpu.sync_copy(dst_hbm.at[pl.ds(r0, 1)], didx_v.at[pl.ds(0, 1)])
        pltpu.async_copy(tg.at[sidx_v.at[0]],
                         rows_v.at[pl.ds(0, LANE)], sem).wait()
        pltpu.async_copy(rows_v.at[pl.ds(0, LANE)],
                         table.at[didx_v.at[0]], sem2, add=True).wait()

    plsc.subcore_barrier()
    pltpu.sync_copy(table.at[pl.ds(s * RPS, RPS)],
                    out_hbm.at[c].at[pl.ds(s * RPS, RPS)])


_sc_gather_scatter = pl.kernel(
    _sc_gs_body,
    out_type=jax.ShapeDtypeStruct((NC, NT, W16), jnp.float32),
    mesh=_MESH,
    scratch_types=[pltpu.VMEM((MROW, LANE), jnp.int32),
                   pltpu.VMEM((MROW, LANE), jnp.int32),
                   pltpu.VMEM((MROW * LANE, W16), jnp.float32),
                   pltpu.VMEM_SHARED((N, W16), jnp.float32),
                   pltpu.VMEM_SHARED((NT, W16), jnp.float32),
                   pltpu.SemaphoreType.DMA,
                   pltpu.SemaphoreType.DMA],
    compiler_params=pltpu.CompilerParams(use_tc_tiling_on_sc=False),
)


def kernel(x, edge_index, edge_attr, We, be, Wn, bn, Wc, bc, Wl, bl, Wo, bo):
    f32 = jnp.float32
    src2 = edge_index[0].astype(jnp.int32).reshape(NROWS, LANE)
    dst2 = edge_index[1].astype(jnp.int32).reshape(NROWS, LANE)

    we16 = jnp.zeros((16, 16), f32).at[:, :8].set(We)
    be16 = jnp.zeros((1, 16), f32).at[0, :8].set(be)
    wct = jnp.zeros((16, 16), f32).at[:8, :10].set(Wc[:8])
    wcb = jnp.zeros((16, 16), f32).at[:, :10].set(Wc[8:])
    bn_r = bn.reshape(1, 16)
    bc_r = jnp.zeros((1, 16), f32).at[0, :10].set(bc)
    wl_r = jnp.zeros((16, 16), f32).at[:10, :10].set(Wl)
    bl_r = jnp.zeros((1, 16), f32).at[0, :10].set(bl)
    wo_r = jnp.zeros((16, 2), f32).at[:10].set(Wo)
    bo_r = bo.reshape(1, 2)
    zeros_t = jnp.zeros((NT, W16), f32)
    ones_b = jnp.ones((LANE, W16), f32)

    npart = _front(x, Wn, bn_r, wcb)
    accp, degp = _sc_scatter(edge_attr, dst2, ones_b, zeros_t)
    gsc = _mid(accp[0, :N], accp[1, :N], degp[0, :N], degp[1, :N],
               npart, we16, be16, wct)
    aggp = _sc_gather_scatter(gsc, src2, dst2, zeros_t)
    return _head(aggp[0, :N], aggp[1, :N], bc_r, wl_r, bl_r, wo_r, bo_r)


# speedup vs baseline: 9.9633x; 1.0620x over previous
import jax
import jax.numpy as jnp
from jax import lax
from jax.experimental import pallas as pl
from jax.experimental.pallas import tpu as pltpu
from jax.experimental.pallas import tpu_sc as plsc

N = 10000
E = 320000
NC, NS = 2, 16
NW = NC * NS
LANE = 128
MROW = 6
NROWS = E // LANE
RPW = 78
NMACRO = RPW // MROW
XBASE = NW * RPW
NT = 10240
RPS = NT // NS
W16 = 16


def _front_body(x_ref, eat_ref, wn_ref, bn_ref, wcb_ref, we_ref, brow_ref,
                np_ref, em_ref):
    t = jnp.maximum(jnp.dot(x_ref[...], wn_ref[...],
                            preferred_element_type=jnp.float32) + bn_ref[...], 0.0)
    np_ref[...] = jnp.dot(t, wcb_ref[...], preferred_element_type=jnp.float32)
    ea = jnp.transpose(eat_ref[...])
    em = jnp.dot(ea, we_ref[...],
                 preferred_element_type=jnp.float32) + brow_ref[...]
    c8 = em.shape[0] // 8
    em_ref[...] = jnp.concatenate([em[k * c8:(k + 1) * c8, :]
                                   for k in range(8)], axis=1)


_CH = E // 10


def _front(x, eat, wn, bn_r, wcb, we16, brow):
    return pl.pallas_call(
        _front_body,
        grid=(10,),
        in_specs=[pl.BlockSpec((1000, 128), lambda i: (i, 0)),
                  pl.BlockSpec((16, _CH), lambda i: (0, i)),
                  pl.BlockSpec((128, 16), lambda i: (0, 0)),
                  pl.BlockSpec((1, 16), lambda i: (0, 0)),
                  pl.BlockSpec((16, 16), lambda i: (0, 0)),
                  pl.BlockSpec((16, 16), lambda i: (0, 0)),
                  pl.BlockSpec((1, 16), lambda i: (0, 0))],
        out_specs=[pl.BlockSpec((1000, 16), lambda i: (i, 0)),
                   pl.BlockSpec((_CH // 8, 128), lambda i: (i, 0))],
        out_shape=[jax.ShapeDtypeStruct((N, W16), jnp.float32),
                   jax.ShapeDtypeStruct((E * W16 // 128, 128), jnp.float32)],
    )(x, eat, wn, bn_r, wcb, we16, brow)


def _mid_body(a0, a1, npart, wct, o):
    acc = a0[...] + a1[...]
    norm = lax.rsqrt(jnp.maximum(acc[:, 8:9], 1.0))
    eh = jnp.maximum(acc, 0.0)
    g = (jnp.dot(eh, wct[...], preferred_element_type=jnp.float32)
         + npart[...]) * norm
    col = lax.broadcasted_iota(jnp.int32, g.shape, 1)
    o[...] = jnp.where(col == 15, 1.0, g)


def _mid(acc0, acc1, npart, wct):
    spec = pl.BlockSpec((1000, 16), lambda i: (i, 0))
    return pl.pallas_call(
        _mid_body,
        grid=(10,),
        in_specs=[spec, spec, spec, pl.BlockSpec((16, 16), lambda i: (0, 0))],
        out_specs=spec,
        out_shape=jax.ShapeDtypeStruct((N, W16), jnp.float32),
    )(acc0, acc1, npart, wct)


def _head_body(g0, g1, bc_r, wl_r, bl_r, wo_r, bo_r, o):
    deg = g0[:, 15:16] + g1[:, 15:16]
    norm = lax.rsqrt(jnp.maximum(deg, 1.0))
    agg = (g0[...] + g1[...]) * norm
    h2 = jnp.maximum(agg + bc_r[...], 0.0)
    h3 = jnp.maximum(jnp.dot(h2, wl_r[...],
                             preferred_element_type=jnp.float32) + bl_r[...], 0.0)
    o[...] = jnp.dot(h3, wo_r[...], preferred_element_type=jnp.float32) + bo_r[...]


def _head(agg0, agg1, bc_r, wl_r, bl_r, wo_r, bo_r):
    spec = pl.BlockSpec((1000, 16), lambda i: (i, 0))
    b16spec = pl.BlockSpec((1, 16), lambda i: (0, 0))
    return pl.pallas_call(
        _head_body,
        grid=(10,),
        in_specs=[spec, spec, b16spec,
                  pl.BlockSpec((16, 16), lambda i: (0, 0)), b16spec,
                  pl.BlockSpec((16, 2), lambda i: (0, 0)),
                  pl.BlockSpec((1, 2), lambda i: (0, 0))],
        out_specs=pl.BlockSpec((1000, 2), lambda i: (i, 0)),
        out_shape=jax.ShapeDtypeStruct((N, 2), jnp.float32),
    )(agg0, agg1, bc_r, wl_r, bl_r, wo_r, bo_r)


_MESH = plsc.VectorSubcoreMesh(core_axis_name="c", subcore_axis_name="s",
                               num_cores=NC, num_subcores=NS)


def _sc_scatter_body(em_hbm, dst_hbm, zero_hbm, out_hbm,
                     idx_v, rows_v, table, sem2):
    c = lax.axis_index("c")
    s = lax.axis_index("s")
    w = c * NS + s

    @pl.when(s == 0)
    def _init():
        pltpu.sync_copy(zero_hbm, table)

    plsc.subcore_barrier()

    def macro(m, carry):
        r0 = w * RPW + m * MROW
        pltpu.sync_copy(dst_hbm.at[pl.ds(r0, MROW)], idx_v)
        pltpu.sync_copy(em_hbm.at[pl.ds(r0 * LANE, MROW * LANE)], rows_v)
        puts = []
        for j in range(MROW):
            puts.append(pltpu.async_copy(rows_v.at[pl.ds(j * LANE, LANE)],
                                         table.at[idx_v.at[j]], sem2,
                                         add=True))
        for p in puts:
            p.wait()
        return carry

    lax.fori_loop(0, NMACRO, macro, 0)

    @pl.when(w < NROWS - XBASE)
    def _leftover():
        r0 = XBASE + w
        pltpu.sync_copy(dst_hbm.at[pl.ds(r0, 1)], idx_v.at[pl.ds(0, 1)])
        pltpu.sync_copy(em_hbm.at[pl.ds(r0 * LANE, LANE)],
                        rows_v.at[pl.ds(0, LANE)])
        pltpu.async_copy(rows_v.at[pl.ds(0, LANE)],
                         table.at[idx_v.at[0]], sem2, add=True).wait()

    plsc.subcore_barrier()
    pltpu.sync_copy(table.at[pl.ds(s * RPS, RPS)],
                    out_hbm.at[c].at[pl.ds(s * RPS, RPS)])


_sc_scatter = pl.kernel(
    _sc_scatter_body,
    out_type=jax.ShapeDtypeStruct((NC, NT, W16), jnp.float32),
    mesh=_MESH,
    scratch_types=[pltpu.VMEM((MROW, LANE), jnp.int32),
                   pltpu.VMEM((MROW * LANE, W16), jnp.float32),
                   pltpu.VMEM_SHARED((NT, W16), jnp.float32),
                   pltpu.SemaphoreType.DMA],
    compiler_params=pltpu.CompilerParams(use_tc_tiling_on_sc=False),
)


def _sc_gs_body(g_hbm, src_hbm, dst_hbm, zero_hbm, out_hbm,
                sidx_v, didx_v, rows_v, tg, table, sem, sem2):
    c = lax.axis_index("c")
    s = lax.axis_index("s")
    w = c * NS + s

    @pl.when(s == 0)
    def _init():
        pltpu.sync_copy(zero_hbm, table)

    @pl.when(s < 10)
    def _stage():
        pltpu.sync_copy(g_hbm.at[pl.ds(s * 1000, 1000)],
                        tg.at[pl.ds(s * 1000, 1000)])

    plsc.subcore_barrier()

    def macro(m, carry):
        r0 = w * RPW + m * MROW
        pltpu.sync_copy(src_hbm.at[pl.ds(r0, MROW)], sidx_v)
        pltpu.sync_copy(dst_hbm.at[pl.ds(r0, MROW)], didx_v)
        gets = [pltpu.async_copy(tg.at[sidx_v.at[j]],
                                 rows_v.at[pl.ds(j * LANE, LANE)], sem)
                for j in range(MROW)]
        puts = []
        for j in range(MROW):
            gets[j].wait()
            puts.append(pltpu.async_copy(rows_v.at[pl.ds(j * LANE, LANE)],
                                         table.at[didx_v.at[j]], sem2,
                                         add=True))
        for p in puts:
            p.wait()
        return carry

    lax.fori_loop(0, NMACRO, macro, 0)

    @pl.when(w < NROWS - XBASE)
    def _leftover():
        r0 = XBASE + w
        pltpu.sync_copy(src_hbm.at[pl.ds(r0, 1)], sidx_v.at[pl.ds(0, 1)])
        pltpu.sync_copy(dst_hbm.at[pl.ds(r0, 1)], didx_v.at[pl.ds(0, 1)])
        pltpu.async_copy(tg.at[sidx_v.at[0]],
                         rows_v.at[pl.ds(0, LANE)], sem).wait()
        pltpu.async_copy(rows_v.at[pl.ds(0, LANE)],
                         table.at[didx_v.at[0]], sem2, add=True).wait()

    plsc.subcore_barrier()
    pltpu.sync_copy(table.at[pl.ds(s * RPS, RPS)],
                    out_hbm.at[c].at[pl.ds(s * RPS, RPS)])


_sc_gather_scatter = pl.kernel(
    _sc_gs_body,
    out_type=jax.ShapeDtypeStruct((NC, NT, W16), jnp.float32),
    mesh=_MESH,
    scratch_types=[pltpu.VMEM((MROW, LANE), jnp.int32),
                   pltpu.VMEM((MROW, LANE), jnp.int32),
                   pltpu.VMEM((MROW * LANE, W16), jnp.float32),
                   pltpu.VMEM_SHARED((N, W16), jnp.float32),
                   pltpu.VMEM_SHARED((NT, W16), jnp.float32),
                   pltpu.SemaphoreType.DMA,
                   pltpu.SemaphoreType.DMA],
    compiler_params=pltpu.CompilerParams(use_tc_tiling_on_sc=False),
)


def kernel(x, edge_index, edge_attr, We, be, Wn, bn, Wc, bc, Wl, bl, Wo, bo):
    f32 = jnp.float32
    src2 = edge_index[0].astype(jnp.int32).reshape(NROWS, LANE)
    dst = edge_index[1].astype(jnp.int32)
    dst2 = dst.reshape(NROWS, LANE)
    dst2p = dst.reshape(10, 8, _CH // 8).swapaxes(1, 2).reshape(NROWS, LANE)

    we16 = jnp.zeros((16, 16), f32).at[:, :8].set(We)
    brow = jnp.concatenate([be, jnp.ones((1,), f32),
                            jnp.zeros((7,), f32)]).reshape(1, 16)
    wct = jnp.zeros((16, 16), f32).at[:8, :10].set(Wc[:8])
    wcb = jnp.zeros((16, 16), f32).at[:, :10].set(Wc[8:])
    bn_r = bn.reshape(1, 16)
    bc_r = jnp.zeros((1, 16), f32).at[0, :10].set(bc)
    wl_r = jnp.zeros((16, 16), f32).at[:10, :10].set(Wl)
    bl_r = jnp.zeros((1, 16), f32).at[0, :10].set(bl)
    wo_r = jnp.zeros((16, 2), f32).at[:10].set(Wo)
    bo_r = bo.reshape(1, 2)
    zeros_t = jnp.zeros((NT, W16), f32)

    npart, em128 = _front(x, jnp.transpose(edge_attr), Wn, bn_r, wcb,
                          we16, brow)
    accp = _sc_scatter(em128.reshape(E, W16), dst2p, zeros_t)
    gsc = _mid(accp[0, :N], accp[1, :N], npart, wct)
    aggp = _sc_gather_scatter(gsc, src2, dst2, zeros_t)
    return _head(aggp[0, :N], aggp[1, :N], bc_r, wl_r, bl_r, wo_r, bo_r)
